# baseline (speedup 1.0000x reference)
import jax
import jax.numpy as jnp
from jax import lax
from jax.experimental import pallas as pl
from jax.experimental.pallas import tpu as pltpu
from jax.experimental.pallas import tpu_sc as plsc

E = 8
TOP_K = 2
D_MODEL = 1024
D_FF = 2048
T = 2048
BT = 256
NTB = T // BT
B = 256
NA = T * TOP_K
SPAD = NA + E * B
NB = SPAD // B
CH = 16


def _router_body(x_ref, gate_ref, e0_ref, e1_ref, r0_ref, r1_ref,
                 w0_ref, w1_ref, base_ref, be_ref, cnt_s, tri_s):
    i = pl.program_id(0)

    @pl.when(i == 0)
    def _():
        for e in range(E):
            cnt_s[e] = 0
        r_iota = lax.broadcasted_iota(jnp.int32, (2 * BT, 2 * BT), 0)
        c_iota = lax.broadcasted_iota(jnp.int32, (2 * BT, 2 * BT), 1)
        tri_s[...] = (r_iota > c_iota).astype(jnp.float32)

    @pl.when(i < NTB)
    def _():
        x = x_ref[...]
        logits = jnp.dot(x, gate_ref[...], preferred_element_type=jnp.float32)
        i1 = jnp.argmax(logits, axis=1)
        m1 = jnp.max(logits, axis=1, keepdims=True)
        col = lax.broadcasted_iota(jnp.int32, logits.shape, 1)
        masked = jnp.where(col == i1[:, None], -jnp.inf, logits)
        i2 = jnp.argmax(masked, axis=1)
        m2 = jnp.max(masked, axis=1, keepdims=True)
        e2 = jnp.exp(m2 - m1)
        denom = 1.0 + e2
        wa = (1.0 / denom)[:, 0]
        wb = (e2 / denom)[:, 0]

        c1 = (col == i1[:, None]).astype(jnp.float32)
        c2 = (col == i2[:, None]).astype(jnp.float32)
        c = jnp.concatenate([c1, c2], axis=0)

        prefix = jnp.dot(tri_s[...], c, preferred_element_type=jnp.float32)
        rank_in = jnp.sum(prefix * c, axis=1)
        prior = jnp.zeros((2 * BT,), jnp.float32)
        for e in range(E):
            prior = prior + cnt_s[e].astype(jnp.float32) * c[:, e]
        rank = (rank_in + prior).astype(jnp.int32)

        e0_ref[0, 0, :] = i1.astype(jnp.int32)
        e1_ref[0, 0, :] = i2.astype(jnp.int32)
        r0_ref[0, 0, :] = rank[:BT]
        r1_ref[0, 0, :] = rank[BT:]
        w0_ref[0, 0, :] = wa
        w1_ref[0, 0, :] = wb

        for e in range(E):
            cnt_s[e] = cnt_s[e] + jnp.sum(c[:, e]).astype(jnp.int32)

    @pl.when(i == NTB)
    def _():
        off = jnp.int32(0)
        for e in range(E):
            base_ref[e] = off
            padded = ((cnt_s[e] + (B - 1)) // B) * B
            off = off + padded
        for b in range(NB):
            acc = jnp.int32(-1)
            for e in range(E):
                acc = acc + (base_ref[e] <= b * B).astype(jnp.int32)
            be_ref[b] = acc
        be_ref[NB] = off // B


def _router(inputs, gate_w):
    blk = lambda i: (jnp.minimum(i, NTB - 1), 0, 0)
    out3 = jax.ShapeDtypeStruct((NTB, 1, BT), jnp.int32)
    out3f = jax.ShapeDtypeStruct((NTB, 1, BT), jnp.float32)
    return pl.pallas_call(
        _router_body,
        grid=(NTB + 1,),
        in_specs=[
            pl.BlockSpec((BT, D_MODEL), lambda i: (jnp.minimum(i, NTB - 1), 0)),
            pl.BlockSpec((D_MODEL, E), lambda i: (0, 0)),
        ],
        out_specs=[
            pl.BlockSpec((1, 1, BT), blk),
            pl.BlockSpec((1, 1, BT), blk),
            pl.BlockSpec((1, 1, BT), blk),
            pl.BlockSpec((1, 1, BT), blk),
            pl.BlockSpec((1, 1, BT), blk),
            pl.BlockSpec((1, 1, BT), blk),
            pl.BlockSpec(memory_space=pltpu.SMEM),
            pl.BlockSpec(memory_space=pltpu.SMEM),
        ],
        out_shape=[out3, out3, out3, out3, out3f, out3f,
                   jax.ShapeDtypeStruct((E,), jnp.int32),
                   jax.ShapeDtypeStruct((NB + 1,), jnp.int32)],
        scratch_shapes=[pltpu.SMEM((E,), jnp.int32),
                        pltpu.VMEM((2 * BT, 2 * BT), jnp.float32)],
        compiler_params=pltpu.CompilerParams(
            dimension_semantics=("arbitrary",),
        ),
    )(inputs, gate_w)


_NC, _NS = 2, 16
_NW = _NC * _NS


def _sc_dispatch_body(x_hbm, e_hbm, r_hbm, base_hbm, disp_hbm,
                      base_vm, e_vm, r_vm, idx_vm, tok_vm, rows_vm, sem):
    wid = lax.axis_index("s") * _NC + lax.axis_index("c")
    per_w = NA // _NW
    pltpu.sync_copy(base_hbm, base_vm)

    def chunk(ci, carry):
        start = wid * per_w + ci * CH
        pltpu.sync_copy(e_hbm.at[pl.ds(start, CH)], e_vm)
        pltpu.sync_copy(r_hbm.at[pl.ds(start, CH)], r_vm)
        e_v = e_vm[...]
        base_reg = base_vm[...]
        b_v = jnp.zeros((CH,), jnp.int32)
        for e in range(E):
            b_v = jnp.where(e_v == e, jnp.full((CH,), base_reg[e], jnp.int32), b_v)
        idx_vm[...] = b_v + r_vm[...]
        tok_vm[...] = (start + lax.iota(jnp.int32, CH)) & (T - 1)
        pltpu.async_copy(x_hbm.at[tok_vm], rows_vm, sem).wait()
        pltpu.async_copy(rows_vm, disp_hbm.at[idx_vm], sem).wait()
        return carry

    lax.fori_loop(0, per_w // CH, chunk, 0)


def _sc_dispatch(inputs, e_flat, r_flat, base):
    return pl.kernel(
        _sc_dispatch_body,
        out_type=jax.ShapeDtypeStruct((SPAD, D_MODEL), jnp.float32),
        mesh=plsc.VectorSubcoreMesh(core_axis_name="c", subcore_axis_name="s"),
        scratch_types=[
            pltpu.VMEM((16,), jnp.int32),
            pltpu.VMEM((CH,), jnp.int32),
            pltpu.VMEM((CH,), jnp.int32),
            pltpu.VMEM((CH,), jnp.int32),
            pltpu.VMEM((CH,), jnp.int32),
            pltpu.VMEM((CH, D_MODEL), jnp.float32),
            pltpu.SemaphoreType.DMA,
        ],
    )(inputs, e_flat, r_flat, base)


def _sc_return_body(y_hbm, e_hbm, r_hbm, base_hbm, g_hbm,
                    base_vm, e_vm, r_vm, idx_vm, rows_vm, sem):
    wid = lax.axis_index("s") * _NC + lax.axis_index("c")
    per_w = NA // _NW
    pltpu.sync_copy(base_hbm, base_vm)

    def chunk(ci, carry):
        start = wid * per_w + ci * CH
        pltpu.sync_copy(e_hbm.at[pl.ds(start, CH)], e_vm)
        pltpu.sync_copy(r_hbm.at[pl.ds(start, CH)], r_vm)
        e_v = e_vm[...]
        base_reg = base_vm[...]
        b_v = jnp.zeros((CH,), jnp.int32)
        for e in range(E):
            b_v = jnp.where(e_v == e, jnp.full((CH,), base_reg[e], jnp.int32), b_v)
        idx_vm[...] = b_v + r_vm[...]
        pltpu.async_copy(y_hbm.at[idx_vm], rows_vm, sem).wait()
        pltpu.async_copy(rows_vm, g_hbm.at[pl.ds(start, CH)], sem).wait()
        return carry

    lax.fori_loop(0, per_w // CH, chunk, 0)


def _sc_return(y, e_flat, r_flat, base):
    return pl.kernel(
        _sc_return_body,
        out_type=jax.ShapeDtypeStruct((NA, D_MODEL), jnp.float32),
        mesh=plsc.VectorSubcoreMesh(core_axis_name="c", subcore_axis_name="s"),
        scratch_types=[
            pltpu.VMEM((16,), jnp.int32),
            pltpu.VMEM((CH,), jnp.int32),
            pltpu.VMEM((CH,), jnp.int32),
            pltpu.VMEM((CH,), jnp.int32),
            pltpu.VMEM((CH, D_MODEL), jnp.float32),
            pltpu.SemaphoreType.DMA,
        ],
    )(y, e_flat, r_flat, base)


BFF = D_FF // 2


def _grouped_body(be_ref, x_ref, w1_ref, w3_ref, w2_ref, y_ref):
    b = pl.program_id(0)
    f = pl.program_id(1)

    @pl.when(b < be_ref[NB])
    def _():
        x = x_ref[...]
        a = jnp.dot(x, w1_ref[0], preferred_element_type=jnp.float32)
        g = jnp.dot(x, w3_ref[0], preferred_element_type=jnp.float32)
        h = jax.nn.silu(a) * g
        contrib = jnp.dot(h, w2_ref[0], preferred_element_type=jnp.float32)

        @pl.when(f == 0)
        def _():
            y_ref[...] = contrib

        @pl.when(f == 1)
        def _():
            y_ref[...] += contrib


def _grouped(be, disp, w1, w3, w2):
    grid_spec = pltpu.PrefetchScalarGridSpec(
        num_scalar_prefetch=1,
        grid=(NB, 2),
        in_specs=[
            pl.BlockSpec((B, D_MODEL), lambda b, f, be: (b, 0)),
            pl.BlockSpec((1, D_MODEL, BFF), lambda b, f, be: (be[b], 0, f)),
            pl.BlockSpec((1, D_MODEL, BFF), lambda b, f, be: (be[b], 0, f)),
            pl.BlockSpec((1, BFF, D_MODEL), lambda b, f, be: (be[b], f, 0)),
        ],
        out_specs=pl.BlockSpec((B, D_MODEL), lambda b, f, be: (b, 0)),
    )
    return pl.pallas_call(
        _grouped_body,
        grid_spec=grid_spec,
        out_shape=jax.ShapeDtypeStruct((SPAD, D_MODEL), jnp.float32),
        compiler_params=pltpu.CompilerParams(
            dimension_semantics=("arbitrary", "arbitrary"),
        ),
    )(be, disp, w1, w3, w2)


def _combine_body(g0_ref, g1_ref, w0_ref, w1_ref, out_ref):
    s0 = w0_ref[0, 0, :][:, None]
    s1 = w1_ref[0, 0, :][:, None]
    out_ref[...] = s0 * g0_ref[...] + s1 * g1_ref[...]


def _combine(g, w0, w1):
    return pl.pallas_call(
        _combine_body,
        grid=(NTB,),
        in_specs=[
            pl.BlockSpec((BT, D_MODEL), lambda t: (t, 0)),
            pl.BlockSpec((BT, D_MODEL), lambda t: (t + NTB, 0)),
            pl.BlockSpec((1, 1, BT), lambda t: (t, 0, 0)),
            pl.BlockSpec((1, 1, BT), lambda t: (t, 0, 0)),
        ],
        out_specs=pl.BlockSpec((BT, D_MODEL), lambda t: (t, 0)),
        out_shape=jax.ShapeDtypeStruct((T, D_MODEL), jnp.float32),
    )(g, g, w0, w1)


def kernel(inputs, gate_w, w1, w2, w3):
    e0, e1, r0, r1, wa, wb, base, be = _router(inputs, gate_w)
    e_flat = jnp.concatenate([e0.reshape(T), e1.reshape(T)])
    r_flat = jnp.concatenate([r0.reshape(T), r1.reshape(T)])
    base16 = jnp.pad(base, (0, 16 - E))
    disp = _sc_dispatch(inputs, e_flat, r_flat, base16)
    y = _grouped(be, disp, w1, w3, w2)
    g = _sc_return(y, e_flat, r_flat, base16)
    return _combine(g, wa, wb)


# speedup vs baseline: 1.2245x; 1.2245x over previous
import jax
import jax.numpy as jnp
from jax import lax
from jax.experimental import pallas as pl
from jax.experimental.pallas import tpu as pltpu
from jax.experimental.pallas import tpu_sc as plsc

E = 8
TOP_K = 2
D_MODEL = 1024
D_FF = 2048
T = 2048
BT = 256
NTB = T // BT
B = 128
NA = T * TOP_K
SPAD = NA + E * B
NB = SPAD // B
CH = 16


def _router_body(x_ref, gate_ref, e0_ref, e1_ref, r0_ref, r1_ref,
                 w0_ref, w1_ref, base_ref, be_ref, cnt_s, tri_s):
    i = pl.program_id(0)

    @pl.when(i == 0)
    def _():
        for e in range(E):
            cnt_s[e] = 0
        r_iota = lax.broadcasted_iota(jnp.int32, (2 * BT, 2 * BT), 0)
        c_iota = lax.broadcasted_iota(jnp.int32, (2 * BT, 2 * BT), 1)
        tri_s[...] = (r_iota > c_iota).astype(jnp.float32)

    @pl.when(i < NTB)
    def _():
        x = x_ref[...]
        logits = jnp.dot(x, gate_ref[...], preferred_element_type=jnp.float32)
        i1 = jnp.argmax(logits, axis=1)
        m1 = jnp.max(logits, axis=1, keepdims=True)
        col = lax.broadcasted_iota(jnp.int32, logits.shape, 1)
        masked = jnp.where(col == i1[:, None], -jnp.inf, logits)
        i2 = jnp.argmax(masked, axis=1)
        m2 = jnp.max(masked, axis=1, keepdims=True)
        e2 = jnp.exp(m2 - m1)
        denom = 1.0 + e2
        wa = (1.0 / denom)[:, 0]
        wb = (e2 / denom)[:, 0]

        c1 = (col == i1[:, None]).astype(jnp.float32)
        c2 = (col == i2[:, None]).astype(jnp.float32)
        c = jnp.concatenate([c1, c2], axis=0)

        prefix = jnp.dot(tri_s[...], c, preferred_element_type=jnp.float32)
        rank_in = jnp.sum(prefix * c, axis=1)
        prior = jnp.zeros((2 * BT,), jnp.float32)
        for e in range(E):
            prior = prior + cnt_s[e].astype(jnp.float32) * c[:, e]
        rank = (rank_in + prior).astype(jnp.int32)

        e0_ref[0, 0, :] = i1.astype(jnp.int32)
        e1_ref[0, 0, :] = i2.astype(jnp.int32)
        r0_ref[0, 0, :] = rank[:BT]
        r1_ref[0, 0, :] = rank[BT:]
        w0_ref[0, 0, :] = wa
        w1_ref[0, 0, :] = wb

        for e in range(E):
            cnt_s[e] = cnt_s[e] + jnp.sum(c[:, e]).astype(jnp.int32)

    @pl.when(i == NTB)
    def _():
        off = jnp.int32(0)
        for e in range(E):
            base_ref[e] = off
            padded = ((cnt_s[e] + (B - 1)) // B) * B
            off = off + padded
        for b in range(NB):
            acc = jnp.int32(-1)
            for e in range(E):
                acc = acc + (base_ref[e] <= b * B).astype(jnp.int32)
            be_ref[b] = acc
        be_ref[NB] = off // B


def _router(inputs, gate_w):
    blk = lambda i: (jnp.minimum(i, NTB - 1), 0, 0)
    out3 = jax.ShapeDtypeStruct((NTB, 1, BT), jnp.int32)
    out3f = jax.ShapeDtypeStruct((NTB, 1, BT), jnp.float32)
    return pl.pallas_call(
        _router_body,
        grid=(NTB + 1,),
        in_specs=[
            pl.BlockSpec((BT, D_MODEL), lambda i: (jnp.minimum(i, NTB - 1), 0)),
            pl.BlockSpec((D_MODEL, E), lambda i: (0, 0)),
        ],
        out_specs=[
            pl.BlockSpec((1, 1, BT), blk),
            pl.BlockSpec((1, 1, BT), blk),
            pl.BlockSpec((1, 1, BT), blk),
            pl.BlockSpec((1, 1, BT), blk),
            pl.BlockSpec((1, 1, BT), blk),
            pl.BlockSpec((1, 1, BT), blk),
            pl.BlockSpec(memory_space=pltpu.SMEM),
            pl.BlockSpec(memory_space=pltpu.SMEM),
        ],
        out_shape=[out3, out3, out3, out3, out3f, out3f,
                   jax.ShapeDtypeStruct((E,), jnp.int32),
                   jax.ShapeDtypeStruct((NB + 1,), jnp.int32)],
        scratch_shapes=[pltpu.SMEM((E,), jnp.int32),
                        pltpu.VMEM((2 * BT, 2 * BT), jnp.float32)],
        compiler_params=pltpu.CompilerParams(
            dimension_semantics=("arbitrary",),
        ),
    )(inputs, gate_w)


_NC, _NS = 2, 16
_NW = _NC * _NS


def _sc_dispatch_body(x_hbm, e_hbm, r_hbm, base_hbm, disp_hbm,
                      base_vm, e_vm, r_vm, idx_vm, tok_vm, rows_vm, sem):
    wid = lax.axis_index("s") * _NC + lax.axis_index("c")
    per_w = NA // _NW
    pltpu.sync_copy(base_hbm, base_vm)

    def chunk(ci, carry):
        start = wid * per_w + ci * CH
        pltpu.sync_copy(e_hbm.at[pl.ds(start, CH)], e_vm)
        pltpu.sync_copy(r_hbm.at[pl.ds(start, CH)], r_vm)
        e_v = e_vm[...]
        base_reg = base_vm[...]
        b_v = jnp.zeros((CH,), jnp.int32)
        for e in range(E):
            b_v = jnp.where(e_v == e, jnp.full((CH,), base_reg[e], jnp.int32), b_v)
        idx_vm[...] = b_v + r_vm[...]
        tok_vm[...] = (start + lax.iota(jnp.int32, CH)) & (T - 1)
        pltpu.async_copy(x_hbm.at[tok_vm], rows_vm, sem).wait()
        pltpu.async_copy(rows_vm, disp_hbm.at[idx_vm], sem).wait()
        return carry

    lax.fori_loop(0, per_w // CH, chunk, 0)


def _sc_dispatch(inputs, e_flat, r_flat, base):
    return pl.kernel(
        _sc_dispatch_body,
        out_type=jax.ShapeDtypeStruct((SPAD, D_MODEL), jnp.float32),
        mesh=plsc.VectorSubcoreMesh(core_axis_name="c", subcore_axis_name="s"),
        scratch_types=[
            pltpu.VMEM((16,), jnp.int32),
            pltpu.VMEM((CH,), jnp.int32),
            pltpu.VMEM((CH,), jnp.int32),
            pltpu.VMEM((CH,), jnp.int32),
            pltpu.VMEM((CH,), jnp.int32),
            pltpu.VMEM((CH, D_MODEL), jnp.float32),
            pltpu.SemaphoreType.DMA,
        ],
    )(inputs, e_flat, r_flat, base)


def _sc_return_body(y_hbm, e_hbm, r_hbm, base_hbm, g_hbm,
                    base_vm, e_vm, r_vm, idx_vm, rows_vm, sem):
    wid = lax.axis_index("s") * _NC + lax.axis_index("c")
    per_w = NA // _NW
    pltpu.sync_copy(base_hbm, base_vm)

    def chunk(ci, carry):
        start = wid * per_w + ci * CH
        pltpu.sync_copy(e_hbm.at[pl.ds(start, CH)], e_vm)
        pltpu.sync_copy(r_hbm.at[pl.ds(start, CH)], r_vm)
        e_v = e_vm[...]
        base_reg = base_vm[...]
        b_v = jnp.zeros((CH,), jnp.int32)
        for e in range(E):
            b_v = jnp.where(e_v == e, jnp.full((CH,), base_reg[e], jnp.int32), b_v)
        idx_vm[...] = b_v + r_vm[...]
        pltpu.async_copy(y_hbm.at[idx_vm], rows_vm, sem).wait()
        pltpu.async_copy(rows_vm, g_hbm.at[pl.ds(start, CH)], sem).wait()
        return carry

    lax.fori_loop(0, per_w // CH, chunk, 0)


def _sc_return(y, e_flat, r_flat, base):
    return pl.kernel(
        _sc_return_body,
        out_type=jax.ShapeDtypeStruct((NA, D_MODEL), jnp.float32),
        mesh=plsc.VectorSubcoreMesh(core_axis_name="c", subcore_axis_name="s"),
        scratch_types=[
            pltpu.VMEM((16,), jnp.int32),
            pltpu.VMEM((CH,), jnp.int32),
            pltpu.VMEM((CH,), jnp.int32),
            pltpu.VMEM((CH,), jnp.int32),
            pltpu.VMEM((CH, D_MODEL), jnp.float32),
            pltpu.SemaphoreType.DMA,
        ],
    )(y, e_flat, r_flat, base)


def _grouped_body(be_ref, x_ref, w1_ref, w3_ref, w2_ref, y_ref):
    b = pl.program_id(0)

    @pl.when(b < be_ref[NB])
    def _():
        x = x_ref[...]
        a = jnp.dot(x, w1_ref[0], preferred_element_type=jnp.float32)
        g = jnp.dot(x, w3_ref[0], preferred_element_type=jnp.float32)
        h = jax.nn.silu(a) * g
        y_ref[...] = jnp.dot(h, w2_ref[0], preferred_element_type=jnp.float32)


def _grouped(be, disp, w1, w3, w2):
    grid_spec = pltpu.PrefetchScalarGridSpec(
        num_scalar_prefetch=1,
        grid=(NB,),
        in_specs=[
            pl.BlockSpec((B, D_MODEL), lambda b, be: (b, 0)),
            pl.BlockSpec((1, D_MODEL, D_FF), lambda b, be: (be[b], 0, 0)),
            pl.BlockSpec((1, D_MODEL, D_FF), lambda b, be: (be[b], 0, 0)),
            pl.BlockSpec((1, D_FF, D_MODEL), lambda b, be: (be[b], 0, 0)),
        ],
        out_specs=pl.BlockSpec((B, D_MODEL), lambda b, be: (b, 0)),
    )
    return pl.pallas_call(
        _grouped_body,
        grid_spec=grid_spec,
        out_shape=jax.ShapeDtypeStruct((SPAD, D_MODEL), jnp.float32),
        compiler_params=pltpu.CompilerParams(
            dimension_semantics=("arbitrary",),
        ),
    )(be, disp, w1, w3, w2)


def _combine_body(g0_ref, g1_ref, w0_ref, w1_ref, out_ref):
    s0 = w0_ref[0, 0, :][:, None]
    s1 = w1_ref[0, 0, :][:, None]
    out_ref[...] = s0 * g0_ref[...] + s1 * g1_ref[...]


def _combine(g, w0, w1):
    return pl.pallas_call(
        _combine_body,
        grid=(NTB,),
        in_specs=[
            pl.BlockSpec((BT, D_MODEL), lambda t: (t, 0)),
            pl.BlockSpec((BT, D_MODEL), lambda t: (t + NTB, 0)),
            pl.BlockSpec((1, 1, BT), lambda t: (t, 0, 0)),
            pl.BlockSpec((1, 1, BT), lambda t: (t, 0, 0)),
        ],
        out_specs=pl.BlockSpec((BT, D_MODEL), lambda t: (t, 0)),
        out_shape=jax.ShapeDtypeStruct((T, D_MODEL), jnp.float32),
    )(g, g, w0, w1)


def kernel(inputs, gate_w, w1, w2, w3):
    e0, e1, r0, r1, wa, wb, base, be = _router(inputs, gate_w)
    e_flat = jnp.concatenate([e0.reshape(T), e1.reshape(T)])
    r_flat = jnp.concatenate([r0.reshape(T), r1.reshape(T)])
    base16 = jnp.pad(base, (0, 16 - E))
    disp = _sc_dispatch(inputs, e_flat, r_flat, base16)
    y = _grouped(be, disp, w1, w3, w2)
    g = _sc_return(y, e_flat, r_flat, base16)
    return _combine(g, wa, wb)
